# Initial kernel scaffold; baseline (speedup 1.0000x reference)
import jax
import jax.numpy as jnp
from jax.experimental import pallas as pl


def kernel(x_data, edge_index, bond_dist, W, bias):
    raise NotImplementedError("write your pallas kernel here")


# speedup vs baseline: 20.1879x; 20.1879x over previous
import functools

import jax
import jax.numpy as jnp
from jax import lax
from jax.experimental import pallas as pl
from jax.experimental.pallas import tpu as pltpu
from jax.experimental.pallas import tpu_sc as plsc

S = 128
N = 558
B2 = 4096
DIM = 128
AROWS = 560
ACOLS = 640
AFLAT = AROWS * ACOLS
NTILES = 16
SPS = S // 2
CHUNKS = B2 // NTILES // 128
SLICE = AFLAT // NTILES


def _prep_body(src_ref, dst_ref, bd_ref, w_ref, fidx_ref):
    src = src_ref[...]
    dst = dst_ref[...]
    bd = bd_ref[...]
    neq = (src != dst).astype(jnp.int32)
    b = jnp.sum(neq, axis=1, keepdims=True)
    pos = lax.broadcasted_iota(jnp.int32, (S, B2), 1)
    mask = pos < b
    masked = jnp.where(mask, bd, -jnp.inf)
    m = jnp.max(masked)
    w_ref[...] = jnp.where(mask, bd / m, jnp.zeros_like(bd))
    fidx_ref[...] = dst * ACOLS + src


def _prep(src, dst, bd):
    return pl.pallas_call(
        _prep_body,
        out_shape=(
            jax.ShapeDtypeStruct((S, B2), jnp.float32),
            jax.ShapeDtypeStruct((S, B2), jnp.int32),
        ),
    )(src, dst, bd)


def _sc_body(fidx_hbm, w_hbm, a_hbm, idx_v, w_v, negw_v, zbuf, a_sh):
    c = lax.axis_index("c")
    sid = lax.axis_index("s")

    def _zero(i, carry):
        zbuf[pl.ds(i * 16, 16)] = jnp.zeros((16,), jnp.float32)
        return carry

    lax.fori_loop(0, SLICE // 16, _zero, 0)
    pltpu.sync_copy(zbuf, a_sh.at[pl.ds(sid * SLICE, SLICE)])
    plsc.subcore_barrier()

    def _sample(si, carry):
        s = c * SPS + si
        pltpu.sync_copy(fidx_hbm.at[s, pl.ds(sid * CHUNKS, CHUNKS)], idx_v)
        pltpu.sync_copy(w_hbm.at[s, pl.ds(sid * CHUNKS, CHUNKS)], w_v)
        for j in range(CHUNKS):
            for k in range(128 // 16):
                negw_v[j, pl.ds(k * 16, 16)] = -w_v[j, pl.ds(k * 16, 16)]
        for j in range(CHUNKS):
            pltpu.sync_copy(w_v.at[j], a_sh.at[idx_v.at[j]], add=True)
        plsc.subcore_barrier()
        pltpu.sync_copy(
            a_sh.at[pl.ds(sid * SLICE, SLICE)],
            a_hbm.at[s, pl.ds(sid * SLICE, SLICE)],
        )
        plsc.subcore_barrier()
        for j in range(CHUNKS):
            pltpu.sync_copy(negw_v.at[j], a_sh.at[idx_v.at[j]], add=True)
        return carry

    lax.fori_loop(0, SPS, _sample, 0)


def _sc_scatter(fidx, w):
    mesh = plsc.VectorSubcoreMesh(core_axis_name="c", subcore_axis_name="s")
    kfn = functools.partial(
        pl.kernel,
        mesh=mesh,
        out_type=jax.ShapeDtypeStruct((S, AFLAT), jnp.float32),
        scratch_types=[
            pltpu.VMEM((CHUNKS, 128), jnp.int32),
            pltpu.VMEM((CHUNKS, 128), jnp.float32),
            pltpu.VMEM((CHUNKS, 128), jnp.float32),
            pltpu.VMEM((SLICE,), jnp.float32),
            pltpu.VMEM_SHARED((AFLAT,), jnp.float32),
        ],
    )(_sc_body)
    return kfn(fidx, w)


def _gcn_body(x_ref, a_ref, w_ref, b_ref, o_ref):
    x = x_ref[0]
    a = a_ref[0]
    h = jnp.dot(x, w_ref[...], preferred_element_type=jnp.float32)
    hp = jnp.concatenate([h, jnp.zeros((AROWS - N, DIM), jnp.float32)], axis=0)
    deg = 1.0 + jnp.sum(a, axis=1)
    dis = lax.rsqrt(deg)
    t = hp * dis[:, None]
    tp = jnp.concatenate(
        [t, jnp.zeros((ACOLS - AROWS, DIM), jnp.float32)], axis=0
    )
    z = jnp.dot(a, tp, preferred_element_type=jnp.float32)
    out = dis[:, None] * (z + t) + b_ref[...]
    o_ref[0] = out[:N]


def _gcn_tc(x_data, a, W, bias2d):
    return pl.pallas_call(
        _gcn_body,
        grid=(S,),
        in_specs=[
            pl.BlockSpec((1, N, DIM), lambda i: (i, 0, 0)),
            pl.BlockSpec((1, AROWS, ACOLS), lambda i: (i, 0, 0)),
            pl.BlockSpec((DIM, DIM), lambda i: (0, 0)),
            pl.BlockSpec((1, DIM), lambda i: (0, 0)),
        ],
        out_specs=pl.BlockSpec((1, N, DIM), lambda i: (i, 0, 0)),
        out_shape=jax.ShapeDtypeStruct((S, N, DIM), jnp.float32),
    )(x_data, a, W, bias2d)


@jax.jit
def kernel(x_data, edge_index, bond_dist, W, bias):
    src = edge_index[:, 0, :].astype(jnp.int32)
    dst = edge_index[:, 1, :].astype(jnp.int32)
    w, fidx = _prep(src, dst, bond_dist)
    a_flat = _sc_scatter(
        fidx.reshape(S, B2 // 128, 128), w.reshape(S, B2 // 128, 128)
    )
    a = a_flat.reshape(S, AROWS, ACOLS)
    out = _gcn_tc(x_data, a, W, bias.reshape(1, DIM))
    return out.reshape(S * N, DIM)
